# baseline (speedup 1.0000x reference)
import functools

import numpy as np
import jax
import jax.numpy as jnp
from jax.experimental import pallas as pl
from jax.experimental.pallas import tpu as pltpu

_ANCHORS = np.array(
    [[12., 16.], [19., 36.], [40., 28.], [36., 75.], [76., 55.], [72., 146.],
     [142., 110.], [192., 243.], [459., 401.]], np.float32)
_NUM_CLASSES = 80
_IMG = 608
_EPS = 1e-7
_B, _A, _H, _W, _N = 16, 3, 76, 76, 20
_CH = 5 + _NUM_CLASSES
_SUB = 6
_STRIDE = _IMG / _W
_SC_ANCH = _ANCHORS / _STRIDE
_LVL = _SC_ANCH[_SUB:_SUB + _A]
_NP = 128
_INTERPRET = False


_ATAN_C = (-0.3333314528, 0.1999355085, -0.1420889944, 0.1065626393,
           -0.0752896400, 0.0429096138, -0.0161657367, 0.0028662257)


def _atan_pos(x):
    inv = x > 1.0
    t = jnp.where(inv, 1.0 / jnp.maximum(x, 1e-30), x)
    t2 = t * t
    p = _ATAN_C[7]
    for c in _ATAN_C[6::-1]:
        p = p * t2 + c
    p = t * (1.0 + t2 * p)
    return jnp.where(inv, (np.pi / 2.0) - p, p)


def _fiota(shape, dim):
    return jax.lax.broadcasted_iota(jnp.int32, shape, dim).astype(jnp.float32)


def _bce_sum_terms(p_raw, tgt):
    p = jnp.clip(p_raw, _EPS, 1.0 - _EPS)
    return -(tgt * jnp.log(p) + (1.0 - tgt) * jnp.log(1.0 - p))


def _ciou(b1x, b1y, b1w, b1h, b2x, b2y, b2w, b2h):
    b1x1 = b1x - 0.5 * b1w
    b1x2 = b1x + 0.5 * b1w
    b1y1 = b1y - 0.5 * b1h
    b1y2 = b1y + 0.5 * b1h
    b2x1 = b2x - 0.5 * b2w
    b2x2 = b2x + 0.5 * b2w
    b2y1 = b2y - 0.5 * b2h
    b2y2 = b2y + 0.5 * b2h
    iw = jnp.maximum(jnp.minimum(b1x2, b2x2) - jnp.maximum(b1x1, b2x1), 0.0)
    ih = jnp.maximum(jnp.minimum(b1y2, b2y2) - jnp.maximum(b1y1, b2y1), 0.0)
    inter = iw * ih
    union = b1w * b1h + b2w * b2h - inter
    iou = inter / jnp.maximum(union, 1e-6)
    cd2 = (b1x - b2x) ** 2 + (b1y - b2y) ** 2
    ewx = jnp.maximum(jnp.maximum(b1x2, b2x2) - jnp.minimum(b1x1, b2x1), 0.0)
    ewy = jnp.maximum(jnp.maximum(b1y2, b2y2) - jnp.minimum(b1y1, b2y1), 0.0)
    ediag = ewx ** 2 + ewy ** 2
    ciou = iou - cd2 / jnp.maximum(ediag, 1e-6)
    v = (4.0 / (np.pi ** 2)) * (
        _atan_pos(b1w / jnp.maximum(b1h, 1e-6))
        - _atan_pos(b2w / jnp.maximum(b2h, 1e-6))) ** 2
    alpha = v / jnp.maximum(1.0 - iou + v, 1e-6)
    return ciou - alpha * v


def _grid_body(xr0, xr1, xr2, xr3, xr4, mt, mc, ms, loss_out, np_out):
    b = pl.program_id(0)
    a = pl.program_id(1)
    af = a.astype(jnp.float32)
    xl = xr0[0, 0]
    yl = xr1[0, 0]
    wl = xr2[0, 0]
    hl = xr3[0, 0]
    cl = xr4[0, 0]
    px = jax.nn.sigmoid(xl)
    py = jax.nn.sigmoid(yl)
    conf = jax.nn.sigmoid(cl)
    aw = jnp.where(a == 0, _LVL[0, 0],
                   jnp.where(a == 1, _LVL[1, 0], _LVL[2, 0]))
    ah = jnp.where(a == 0, _LVL[0, 1],
                   jnp.where(a == 1, _LVL[1, 1], _LVL[2, 1]))
    col_i = _fiota((_H, _W), 1)
    row_i = _fiota((_H, _W), 0)
    pbx = px + col_i
    pby = py + row_i
    pbw = jnp.exp(wl) * aw
    pbh = jnp.exp(hl) * ah
    px1 = pbx - 0.5 * pbw
    px2 = pbx + 0.5 * pbw
    py1 = pby - 0.5 * pbh
    py2 = pby + 0.5 * pbh
    area_p = pbw * pbh

    m = mt[0]
    gx_r = m[0:1]
    gy_r = m[1:2]
    gw_r = m[2:3]
    gh_r = m[3:4]
    a_r = m[4:5]
    gi_r = m[5:6]
    gj_r = m[6:7]
    v_r = m[7:8]
    tw_r = m[9:10]
    th_r = m[10:11]
    sel_r = jnp.where(a_r == af, v_r, 0.0)
    mcol = mc[0]
    a_c = mcol[:, 4:5]
    gi_c = mcol[:, 5:6]
    gj_c = mcol[:, 6:7]
    v_c = mcol[:, 7:8]
    sel_c = jnp.where(a_c == af, v_c, 0.0)

    iota_hn = _fiota((_H, _NP), 0)
    amat = jnp.where(iota_hn == gj_r, sel_r, 0.0)
    iota_nw = _fiota((_NP, _W), 1)
    bt = jnp.where(iota_nw == gi_c, 1.0, 0.0)
    obj = jnp.minimum(
        jnp.dot(amat, bt, preferred_element_type=jnp.float32), 1.0)
    npos = jnp.sum(obj)

    ioumax = jnp.zeros((_H, _W), jnp.float32)
    for n in range(_N):
        gxn = ms[b, 0, n]
        gyn = ms[b, 1, n]
        gwn = ms[b, 2, n]
        ghn = ms[b, 3, n]
        bx1 = gxn - 0.5 * gwn
        bx2 = gxn + 0.5 * gwn
        by1 = gyn - 0.5 * ghn
        by2 = gyn + 0.5 * ghn
        area_b = gwn * ghn
        iw = jnp.maximum(jnp.minimum(px2, bx2) - jnp.maximum(px1, bx1), 0.0)
        ih = jnp.maximum(jnp.minimum(py2, by2) - jnp.maximum(py1, by1), 0.0)
        inter = iw * ih
        union = area_p + (area_b - inter)
        ioumax = jnp.maximum(ioumax, inter / jnp.maximum(union, 1e-6))
    no_obj = jnp.where(ioumax > 0.5, 0.0, 1.0 - obj)
    loss_conf = jnp.sum(_bce_sum_terms(conf, obj) * (obj + no_obj))

    iota_wn = _fiota((_W, _NP), 0)
    bcol = jnp.where(iota_wn == gi_r, 1.0, 0.0)
    rowmask = jnp.where(iota_hn == gj_r, 1.0, 0.0)

    def gather(g):
        colsel = jnp.dot(g, bcol, preferred_element_type=jnp.float32)
        return jnp.sum(colsel * rowmask, axis=0, keepdims=True)

    gbx = gather(pbx)
    gby = gather(pby)
    gbw = gather(pbw)
    gbh = gather(pbh)

    same = jnp.where((gi_c == gi_r) & (gj_c == gj_r), sel_c * sel_r, 0.0)
    iota_m = jax.lax.broadcasted_iota(jnp.int32, (_NP, _NP), 0)
    iota_n = jax.lax.broadcasted_iota(jnp.int32, (_NP, _NP), 1)
    later = jnp.where(iota_m > iota_n, same, 0.0)
    overwritten = jnp.max(later, axis=0, keepdims=True)
    owner = sel_r * (1.0 - overwritten)

    ciou = _ciou(gbx, gby, gbw, gbh, gx_r, gy_r, gw_r, gh_r)
    scale = 2.0 - tw_r * th_r
    loss_loc = jnp.sum((1.0 - ciou) * scale * owner)

    val = (loss_conf + loss_loc) * (1.0 / 1024.0)
    loss_out[...] = jnp.full((1, 1, 8, 128), 0.0, jnp.float32) + val
    np_out[...] = jnp.full((1, 1, 8, 128), 0.0, jnp.float32) + npos * (1.0 / 1024.0)


def _cls_body(pf, xt_hbm, mt, mc, out, buf, sems):
    b = pl.program_id(0)

    def _issue(slot, bb):
        for n in range(_N):
            gj = pf[1, bb, n]
            gi = pf[2, bb, n]
            pltpu.make_async_copy(
                xt_hbm.at[gj, gi],
                buf.at[slot, n],
                sems.at[slot, n]).start()

    @pl.when(b == 0)
    def _():
        _issue(0, 0)

    @pl.when(b + 1 < _B)
    def _():
        _issue((b + 1) & 1, b + 1)

    m = mt[0]
    a_r = m[4:5]
    gi_r = m[5:6]
    gj_r = m[6:7]
    v_r = m[7:8]
    mcol = mc[0]
    a_c = mcol[:, 4:5]
    gi_c = mcol[:, 5:6]
    gj_c = mcol[:, 6:7]
    v_c = mcol[:, 7:8]

    same = jnp.where((a_c == a_r) & (gi_c == gi_r) & (gj_c == gj_r),
                     v_c * v_r, 0.0)
    iota_m = jax.lax.broadcasted_iota(jnp.int32, (_NP, _NP), 0)
    iota_n = jax.lax.broadcasted_iota(jnp.int32, (_NP, _NP), 1)
    later_t = jnp.where(iota_n > iota_m, same, 0.0)
    ow_c = jnp.max(later_t, axis=1, keepdims=True)
    owner_c = jnp.where(v_c > 0.0, 1.0 - ow_c, 0.0)

    same32 = same[0:32, 0:32]
    a32 = mcol[0:32, 4:5]
    cls32 = mcol[0:32, 8:9]
    ch32 = _CH * a32 + 5.0 + cls32
    iota_c = _fiota((32, 255), 1)
    oh = jnp.where(iota_c == ch32, 1.0, 0.0)
    cnt = jnp.dot(same32, oh, preferred_element_type=jnp.float32)
    multihot = jnp.minimum(cnt, 1.0)
    lo = _CH * a32 + 5.0
    chwin = jnp.where((iota_c >= lo) & (iota_c < lo + 80.0), 1.0, 0.0)

    iota_b = _fiota((_B, 1), 0)
    bmask = jnp.where(iota_b == b.astype(jnp.float32), 1.0, 0.0)
    slot = b & 1
    total = 0.0
    for n in range(_N):
        pltpu.make_async_copy(buf.at[slot, n], buf.at[slot, n],
                              sems.at[slot, n]).wait()
        blk = buf[slot, n]
        row = jnp.sum(blk * bmask, axis=0, keepdims=True)
        bce = _bce_sum_terms(jax.nn.sigmoid(row), multihot[n:n + 1])
        total = total + jnp.sum(bce * chwin[n:n + 1]) * owner_c[n, 0]
    out[...] = jnp.full((1, 8, 128), 0.0, jnp.float32) + total * (1.0 / 1024.0)


def _final_body(l1, n1, l2, loss_out, np_out):
    loss = jnp.sum(l1[...]) + jnp.sum(l2[...])
    npos = jnp.maximum(jnp.sum(n1[...]), 1.0)
    loss_out[...] = jnp.zeros((8, 128), jnp.float32) + loss
    np_out[...] = jnp.zeros((8, 128), jnp.float32) + npos


def kernel(input, targets):
    x = input
    t = targets.astype(jnp.float32)
    B, A, H, W, N = _B, _A, _H, _W, _N

    gx = t[..., 0] * W
    gy = t[..., 1] * H
    gw = t[..., 2] * W
    gh = t[..., 3] * H
    gi = jnp.floor(gx).astype(jnp.int32)
    gj = jnp.floor(gy).astype(jnp.int32)
    anw = jnp.asarray(_SC_ANCH[:, 0])
    anh = jnp.asarray(_SC_ANCH[:, 1])
    inter = jnp.minimum(gw[..., None], anw) * jnp.minimum(gh[..., None], anh)
    union = (gw * gh)[..., None] + anw * anh - inter
    best = jnp.argmax(inter / jnp.maximum(union, 1e-6), axis=-1)
    valid = (best >= _SUB) & (best < _SUB + A) & (gj < H) & (gi < W)
    a_idx = jnp.where(valid, best - _SUB, A).astype(jnp.int32)

    meta20 = jnp.stack([
        gx, gy, gw, gh,
        a_idx.astype(jnp.float32),
        gi.astype(jnp.float32), gj.astype(jnp.float32),
        valid.astype(jnp.float32),
        t[..., 4], t[..., 2], t[..., 3],
        (gj & 7).astype(jnp.float32),
    ], axis=1)
    meta_t = jnp.pad(meta20, ((0, 0), (0, 0), (0, _NP - N)))
    meta_c = jnp.transpose(meta_t, (0, 2, 1))

    a_safe = jnp.clip(a_idx, 0, A - 1)
    gj_safe = jnp.clip(gj, 0, H - 1)
    gi_safe = jnp.clip(gi, 0, W - 1)
    pf = jnp.stack([a_safe, gj_safe, gi_safe],
                   axis=0).astype(jnp.int32)
    xt = jnp.transpose(x, (2, 3, 0, 1))
    xs = [x[:, k::_CH, :, :] for k in range(5)]

    loss_p, np_p = pl.pallas_call(
        _grid_body,
        grid=(B, A),
        in_specs=[
            pl.BlockSpec((1, 1, H, W), lambda b, a: (b, a, 0, 0)),
            pl.BlockSpec((1, 1, H, W), lambda b, a: (b, a, 0, 0)),
            pl.BlockSpec((1, 1, H, W), lambda b, a: (b, a, 0, 0)),
            pl.BlockSpec((1, 1, H, W), lambda b, a: (b, a, 0, 0)),
            pl.BlockSpec((1, 1, H, W), lambda b, a: (b, a, 0, 0)),
            pl.BlockSpec((1, 12, _NP), lambda b, a: (b, 0, 0)),
            pl.BlockSpec((1, _NP, 12), lambda b, a: (b, 0, 0)),
            pl.BlockSpec(memory_space=pltpu.SMEM),
        ],
        out_specs=[
            pl.BlockSpec((1, 1, 8, 128), lambda b, a: (b, a, 0, 0)),
            pl.BlockSpec((1, 1, 8, 128), lambda b, a: (b, a, 0, 0)),
        ],
        out_shape=[
            jax.ShapeDtypeStruct((B, A, 8, 128), jnp.float32),
            jax.ShapeDtypeStruct((B, A, 8, 128), jnp.float32),
        ],
        compiler_params=pltpu.CompilerParams(
            dimension_semantics=("parallel", "arbitrary")),
        name="yolo_grid_pass",
        interpret=_INTERPRET,
    )(*xs, meta_t, meta_c, meta20)

    cls_p = pl.pallas_call(
        _cls_body,
        grid=(B,),
        in_specs=[
            pl.BlockSpec(memory_space=pltpu.SMEM),
            pl.BlockSpec(memory_space=pl.ANY),
            pl.BlockSpec((1, 12, _NP), lambda b: (b, 0, 0)),
            pl.BlockSpec((1, _NP, 12), lambda b: (b, 0, 0)),
        ],
        out_specs=pl.BlockSpec((1, 8, 128), lambda b: (b, 0, 0)),
        scratch_shapes=[
            pltpu.VMEM((2, _N, _B, 255), jnp.float32),
            pltpu.SemaphoreType.DMA((2, _N)),
        ],
        out_shape=jax.ShapeDtypeStruct((B, 8, 128), jnp.float32),
        compiler_params=pltpu.CompilerParams(
            dimension_semantics=("arbitrary",)),
        name="yolo_cls_pass",
        interpret=_INTERPRET,
    )(pf, xt, meta_t, meta_c)

    loss_o, np_o = pl.pallas_call(
        _final_body,
        out_shape=[
            jax.ShapeDtypeStruct((8, 128), jnp.float32),
            jax.ShapeDtypeStruct((8, 128), jnp.float32),
        ],
        name="yolo_finalize",
        interpret=_INTERPRET,
    )(loss_p, np_p, cls_p)

    return loss_o[0, 0], np_o[0, 0]


# speedup vs baseline: 7.6329x; 7.6329x over previous
import functools

import numpy as np
import jax
import jax.numpy as jnp
from jax.experimental import pallas as pl
from jax.experimental.pallas import tpu as pltpu

_ANCHORS = np.array(
    [[12., 16.], [19., 36.], [40., 28.], [36., 75.], [76., 55.], [72., 146.],
     [142., 110.], [192., 243.], [459., 401.]], np.float32)
_NUM_CLASSES = 80
_IMG = 608
_EPS = 1e-7
_B, _A, _H, _W, _N = 16, 3, 76, 76, 20
_CH = 5 + _NUM_CLASSES
_SUB = 6
_STRIDE = _IMG / _W
_SC_ANCH = _ANCHORS / _STRIDE
_LVL = _SC_ANCH[_SUB:_SUB + _A]
_NP = 128
_INTERPRET = False


_ATAN_C = (-0.3333314528, 0.1999355085, -0.1420889944, 0.1065626393,
           -0.0752896400, 0.0429096138, -0.0161657367, 0.0028662257)


def _atan_pos(x):
    inv = x > 1.0
    t = jnp.where(inv, 1.0 / jnp.maximum(x, 1e-30), x)
    t2 = t * t
    p = _ATAN_C[7]
    for c in _ATAN_C[6::-1]:
        p = p * t2 + c
    p = t * (1.0 + t2 * p)
    return jnp.where(inv, (np.pi / 2.0) - p, p)


def _fiota(shape, dim):
    return jax.lax.broadcasted_iota(jnp.int32, shape, dim).astype(jnp.float32)


def _bce_sum_terms(p_raw, tgt):
    p = jnp.clip(p_raw, _EPS, 1.0 - _EPS)
    return -(tgt * jnp.log(p) + (1.0 - tgt) * jnp.log(1.0 - p))


def _ciou(b1x, b1y, b1w, b1h, b2x, b2y, b2w, b2h):
    b1x1 = b1x - 0.5 * b1w
    b1x2 = b1x + 0.5 * b1w
    b1y1 = b1y - 0.5 * b1h
    b1y2 = b1y + 0.5 * b1h
    b2x1 = b2x - 0.5 * b2w
    b2x2 = b2x + 0.5 * b2w
    b2y1 = b2y - 0.5 * b2h
    b2y2 = b2y + 0.5 * b2h
    iw = jnp.maximum(jnp.minimum(b1x2, b2x2) - jnp.maximum(b1x1, b2x1), 0.0)
    ih = jnp.maximum(jnp.minimum(b1y2, b2y2) - jnp.maximum(b1y1, b2y1), 0.0)
    inter = iw * ih
    union = b1w * b1h + b2w * b2h - inter
    iou = inter / jnp.maximum(union, 1e-6)
    cd2 = (b1x - b2x) ** 2 + (b1y - b2y) ** 2
    ewx = jnp.maximum(jnp.maximum(b1x2, b2x2) - jnp.minimum(b1x1, b2x1), 0.0)
    ewy = jnp.maximum(jnp.maximum(b1y2, b2y2) - jnp.minimum(b1y1, b2y1), 0.0)
    ediag = ewx ** 2 + ewy ** 2
    ciou = iou - cd2 / jnp.maximum(ediag, 1e-6)
    v = (4.0 / (np.pi ** 2)) * (
        _atan_pos(b1w / jnp.maximum(b1h, 1e-6))
        - _atan_pos(b2w / jnp.maximum(b2h, 1e-6))) ** 2
    alpha = v / jnp.maximum(1.0 - iou + v, 1e-6)
    return ciou - alpha * v


def _grid_body(x5, mt, mc, ms, loss_out, np_out):
    b = pl.program_id(0)
    a = pl.program_id(1)
    af = a.astype(jnp.float32)
    xl = x5[0, 0]
    yl = x5[0, 1]
    wl = x5[0, 2]
    hl = x5[0, 3]
    cl = x5[0, 4]
    px = jax.nn.sigmoid(xl)
    py = jax.nn.sigmoid(yl)
    conf = jax.nn.sigmoid(cl)
    aw = jnp.where(a == 0, _LVL[0, 0],
                   jnp.where(a == 1, _LVL[1, 0], _LVL[2, 0]))
    ah = jnp.where(a == 0, _LVL[0, 1],
                   jnp.where(a == 1, _LVL[1, 1], _LVL[2, 1]))
    col_i = _fiota((_H, _W), 1)
    row_i = _fiota((_H, _W), 0)
    pbx = px + col_i
    pby = py + row_i
    pbw = jnp.exp(wl) * aw
    pbh = jnp.exp(hl) * ah
    px1 = pbx - 0.5 * pbw
    px2 = pbx + 0.5 * pbw
    py1 = pby - 0.5 * pbh
    py2 = pby + 0.5 * pbh
    area_p = pbw * pbh

    m = mt[0]
    gx_r = m[0:1]
    gy_r = m[1:2]
    gw_r = m[2:3]
    gh_r = m[3:4]
    a_r = m[4:5]
    gi_r = m[5:6]
    gj_r = m[6:7]
    v_r = m[7:8]
    tw_r = m[9:10]
    th_r = m[10:11]
    sel_r = jnp.where(a_r == af, v_r, 0.0)
    mcol = mc[0]
    a_c = mcol[:, 4:5]
    gi_c = mcol[:, 5:6]
    gj_c = mcol[:, 6:7]
    v_c = mcol[:, 7:8]
    sel_c = jnp.where(a_c == af, v_c, 0.0)

    iota_hn = _fiota((_H, _NP), 0)
    amat = jnp.where(iota_hn == gj_r, sel_r, 0.0)
    iota_nw = _fiota((_NP, _W), 1)
    bt = jnp.where(iota_nw == gi_c, 1.0, 0.0)
    obj = jnp.minimum(
        jnp.dot(amat, bt, preferred_element_type=jnp.float32), 1.0)
    npos = jnp.sum(obj)

    ioumax = jnp.zeros((_H, _W), jnp.float32)
    for n in range(_N):
        gxn = ms[b, 0, n]
        gyn = ms[b, 1, n]
        gwn = ms[b, 2, n]
        ghn = ms[b, 3, n]
        bx1 = gxn - 0.5 * gwn
        bx2 = gxn + 0.5 * gwn
        by1 = gyn - 0.5 * ghn
        by2 = gyn + 0.5 * ghn
        area_b = gwn * ghn
        iw = jnp.maximum(jnp.minimum(px2, bx2) - jnp.maximum(px1, bx1), 0.0)
        ih = jnp.maximum(jnp.minimum(py2, by2) - jnp.maximum(py1, by1), 0.0)
        inter = iw * ih
        union = area_p + (area_b - inter)
        ioumax = jnp.maximum(ioumax, inter / jnp.maximum(union, 1e-6))
    no_obj = jnp.where(ioumax > 0.5, 0.0, 1.0 - obj)
    loss_conf = jnp.sum(_bce_sum_terms(conf, obj) * (obj + no_obj))

    iota_wn = _fiota((_W, _NP), 0)
    bcol = jnp.where(iota_wn == gi_r, 1.0, 0.0)
    rowmask = jnp.where(iota_hn == gj_r, 1.0, 0.0)

    def gather(g):
        colsel = jnp.dot(g, bcol, preferred_element_type=jnp.float32)
        return jnp.sum(colsel * rowmask, axis=0, keepdims=True)

    gbx = gather(pbx)
    gby = gather(pby)
    gbw = gather(pbw)
    gbh = gather(pbh)

    same = jnp.where((gi_c == gi_r) & (gj_c == gj_r), sel_c * sel_r, 0.0)
    iota_m = jax.lax.broadcasted_iota(jnp.int32, (_NP, _NP), 0)
    iota_n = jax.lax.broadcasted_iota(jnp.int32, (_NP, _NP), 1)
    later = jnp.where(iota_m > iota_n, same, 0.0)
    overwritten = jnp.max(later, axis=0, keepdims=True)
    owner = sel_r * (1.0 - overwritten)

    ciou = _ciou(gbx, gby, gbw, gbh, gx_r, gy_r, gw_r, gh_r)
    scale = 2.0 - tw_r * th_r
    loss_loc = jnp.sum((1.0 - ciou) * scale * owner)

    val = (loss_conf + loss_loc) * (1.0 / 1024.0)
    loss_out[...] = jnp.full((1, 1, 8, 128), 0.0, jnp.float32) + val
    np_out[...] = jnp.full((1, 1, 8, 128), 0.0, jnp.float32) + npos * (1.0 / 1024.0)


def _cls_body(pf, xt_hbm, mt, mc, out, buf, sems):
    b = pl.program_id(0)

    def _issue(slot, bb):
        for n in range(_N):
            gj = pf[1, bb, n]
            gi = pf[2, bb, n]
            pltpu.make_async_copy(
                xt_hbm.at[gj, gi],
                buf.at[slot, n],
                sems.at[slot, n]).start()

    @pl.when(b == 0)
    def _():
        _issue(0, 0)

    @pl.when(b + 1 < _B)
    def _():
        _issue((b + 1) & 1, b + 1)

    m = mt[0]
    a_r = m[4:5]
    gi_r = m[5:6]
    gj_r = m[6:7]
    v_r = m[7:8]
    mcol = mc[0]
    a_c = mcol[:, 4:5]
    gi_c = mcol[:, 5:6]
    gj_c = mcol[:, 6:7]
    v_c = mcol[:, 7:8]

    same = jnp.where((a_c == a_r) & (gi_c == gi_r) & (gj_c == gj_r),
                     v_c * v_r, 0.0)
    iota_m = jax.lax.broadcasted_iota(jnp.int32, (_NP, _NP), 0)
    iota_n = jax.lax.broadcasted_iota(jnp.int32, (_NP, _NP), 1)
    later_t = jnp.where(iota_n > iota_m, same, 0.0)
    ow_c = jnp.max(later_t, axis=1, keepdims=True)
    owner_c = jnp.where(v_c > 0.0, 1.0 - ow_c, 0.0)

    same32 = same[0:32, 0:32]
    a32 = mcol[0:32, 4:5]
    cls32 = mcol[0:32, 8:9]
    ch32 = _CH * a32 + 5.0 + cls32
    iota_c = _fiota((32, 255), 1)
    oh = jnp.where(iota_c == ch32, 1.0, 0.0)
    cnt = jnp.dot(same32, oh, preferred_element_type=jnp.float32)
    multihot = jnp.minimum(cnt, 1.0)
    lo = _CH * a32 + 5.0
    chwin = jnp.where((iota_c >= lo) & (iota_c < lo + 80.0), 1.0, 0.0)

    iota_b = _fiota((_B, 1), 0)
    bmask = jnp.where(iota_b == b.astype(jnp.float32), 1.0, 0.0)
    slot = b & 1
    total = 0.0
    for n in range(_N):
        pltpu.make_async_copy(buf.at[slot, n], buf.at[slot, n],
                              sems.at[slot, n]).wait()
        blk = buf[slot, n]
        row = jnp.sum(blk * bmask, axis=0, keepdims=True)
        bce = _bce_sum_terms(jax.nn.sigmoid(row), multihot[n:n + 1])
        total = total + jnp.sum(bce * chwin[n:n + 1]) * owner_c[n, 0]
    out[...] = jnp.full((1, 8, 128), 0.0, jnp.float32) + total * (1.0 / 1024.0)


def _final_body(l1, n1, l2, loss_out, np_out):
    loss = jnp.sum(l1[...]) + jnp.sum(l2[...])
    npos = jnp.maximum(jnp.sum(n1[...]), 1.0)
    loss_out[...] = jnp.zeros((8, 128), jnp.float32) + loss
    np_out[...] = jnp.zeros((8, 128), jnp.float32) + npos


def kernel(input, targets):
    x = input
    t = targets.astype(jnp.float32)
    B, A, H, W, N = _B, _A, _H, _W, _N

    gx = t[..., 0] * W
    gy = t[..., 1] * H
    gw = t[..., 2] * W
    gh = t[..., 3] * H
    gi = jnp.floor(gx).astype(jnp.int32)
    gj = jnp.floor(gy).astype(jnp.int32)
    anw = jnp.asarray(_SC_ANCH[:, 0])
    anh = jnp.asarray(_SC_ANCH[:, 1])
    inter = jnp.minimum(gw[..., None], anw) * jnp.minimum(gh[..., None], anh)
    union = (gw * gh)[..., None] + anw * anh - inter
    best = jnp.argmax(inter / jnp.maximum(union, 1e-6), axis=-1)
    valid = (best >= _SUB) & (best < _SUB + A) & (gj < H) & (gi < W)
    a_idx = jnp.where(valid, best - _SUB, A).astype(jnp.int32)

    meta20 = jnp.stack([
        gx, gy, gw, gh,
        a_idx.astype(jnp.float32),
        gi.astype(jnp.float32), gj.astype(jnp.float32),
        valid.astype(jnp.float32),
        t[..., 4], t[..., 2], t[..., 3],
        (gj & 7).astype(jnp.float32),
    ], axis=1)
    meta_t = jnp.pad(meta20, ((0, 0), (0, 0), (0, _NP - N)))
    meta_c = jnp.transpose(meta_t, (0, 2, 1))

    a_safe = jnp.clip(a_idx, 0, A - 1)
    gj_safe = jnp.clip(gj, 0, H - 1)
    gi_safe = jnp.clip(gi, 0, W - 1)
    pf = jnp.stack([a_safe, gj_safe, gi_safe],
                   axis=0).astype(jnp.int32)
    xt = jnp.transpose(x, (2, 3, 0, 1))

    loss_p, np_p = pl.pallas_call(
        _grid_body,
        grid=(B, A),
        in_specs=[
            pl.BlockSpec((1, 5, H, W), lambda b, a: (b, 17 * a, 0, 0)),
            pl.BlockSpec((1, 12, _NP), lambda b, a: (b, 0, 0)),
            pl.BlockSpec((1, _NP, 12), lambda b, a: (b, 0, 0)),
            pl.BlockSpec(memory_space=pltpu.SMEM),
        ],
        out_specs=[
            pl.BlockSpec((1, 1, 8, 128), lambda b, a: (b, a, 0, 0)),
            pl.BlockSpec((1, 1, 8, 128), lambda b, a: (b, a, 0, 0)),
        ],
        out_shape=[
            jax.ShapeDtypeStruct((B, A, 8, 128), jnp.float32),
            jax.ShapeDtypeStruct((B, A, 8, 128), jnp.float32),
        ],
        compiler_params=pltpu.CompilerParams(
            dimension_semantics=("parallel", "arbitrary")),
        name="yolo_grid_pass",
        interpret=_INTERPRET,
    )(x, meta_t, meta_c, meta20)

    cls_p = pl.pallas_call(
        _cls_body,
        grid=(B,),
        in_specs=[
            pl.BlockSpec(memory_space=pltpu.SMEM),
            pl.BlockSpec(memory_space=pl.ANY),
            pl.BlockSpec((1, 12, _NP), lambda b: (b, 0, 0)),
            pl.BlockSpec((1, _NP, 12), lambda b: (b, 0, 0)),
        ],
        out_specs=pl.BlockSpec((1, 8, 128), lambda b: (b, 0, 0)),
        scratch_shapes=[
            pltpu.VMEM((2, _N, _B, 255), jnp.float32),
            pltpu.SemaphoreType.DMA((2, _N)),
        ],
        out_shape=jax.ShapeDtypeStruct((B, 8, 128), jnp.float32),
        compiler_params=pltpu.CompilerParams(
            dimension_semantics=("arbitrary",)),
        name="yolo_cls_pass",
        interpret=_INTERPRET,
    )(pf, xt, meta_t, meta_c)

    loss_o, np_o = pl.pallas_call(
        _final_body,
        out_shape=[
            jax.ShapeDtypeStruct((8, 128), jnp.float32),
            jax.ShapeDtypeStruct((8, 128), jnp.float32),
        ],
        name="yolo_finalize",
        interpret=_INTERPRET,
    )(loss_p, np_p, cls_p)

    return loss_o[0, 0], np_o[0, 0]


# speedup vs baseline: 7.6749x; 1.0055x over previous
import functools

import numpy as np
import jax
import jax.numpy as jnp
from jax.experimental import pallas as pl
from jax.experimental.pallas import tpu as pltpu

_ANCHORS = np.array(
    [[12., 16.], [19., 36.], [40., 28.], [36., 75.], [76., 55.], [72., 146.],
     [142., 110.], [192., 243.], [459., 401.]], np.float32)
_NUM_CLASSES = 80
_IMG = 608
_EPS = 1e-7
_B, _A, _H, _W, _N = 16, 3, 76, 76, 20
_CH = 5 + _NUM_CLASSES
_SUB = 6
_STRIDE = _IMG / _W
_SC_ANCH = _ANCHORS / _STRIDE
_LVL = _SC_ANCH[_SUB:_SUB + _A]
_NP = 128
_INTERPRET = False


_ATAN_C = (-0.3333314528, 0.1999355085, -0.1420889944, 0.1065626393,
           -0.0752896400, 0.0429096138, -0.0161657367, 0.0028662257)


def _atan_pos(x):
    inv = x > 1.0
    t = jnp.where(inv, 1.0 / jnp.maximum(x, 1e-30), x)
    t2 = t * t
    p = _ATAN_C[7]
    for c in _ATAN_C[6::-1]:
        p = p * t2 + c
    p = t * (1.0 + t2 * p)
    return jnp.where(inv, (np.pi / 2.0) - p, p)


def _fiota(shape, dim):
    return jax.lax.broadcasted_iota(jnp.int32, shape, dim).astype(jnp.float32)


def _bce_sum_terms(p_raw, tgt):
    p = jnp.clip(p_raw, _EPS, 1.0 - _EPS)
    return -(tgt * jnp.log(p) + (1.0 - tgt) * jnp.log(1.0 - p))


def _ciou(b1x, b1y, b1w, b1h, b2x, b2y, b2w, b2h):
    b1x1 = b1x - 0.5 * b1w
    b1x2 = b1x + 0.5 * b1w
    b1y1 = b1y - 0.5 * b1h
    b1y2 = b1y + 0.5 * b1h
    b2x1 = b2x - 0.5 * b2w
    b2x2 = b2x + 0.5 * b2w
    b2y1 = b2y - 0.5 * b2h
    b2y2 = b2y + 0.5 * b2h
    iw = jnp.maximum(jnp.minimum(b1x2, b2x2) - jnp.maximum(b1x1, b2x1), 0.0)
    ih = jnp.maximum(jnp.minimum(b1y2, b2y2) - jnp.maximum(b1y1, b2y1), 0.0)
    inter = iw * ih
    union = b1w * b1h + b2w * b2h - inter
    iou = inter / jnp.maximum(union, 1e-6)
    cd2 = (b1x - b2x) ** 2 + (b1y - b2y) ** 2
    ewx = jnp.maximum(jnp.maximum(b1x2, b2x2) - jnp.minimum(b1x1, b2x1), 0.0)
    ewy = jnp.maximum(jnp.maximum(b1y2, b2y2) - jnp.minimum(b1y1, b2y1), 0.0)
    ediag = ewx ** 2 + ewy ** 2
    ciou = iou - cd2 / jnp.maximum(ediag, 1e-6)
    v = (4.0 / (np.pi ** 2)) * (
        _atan_pos(b1w / jnp.maximum(b1h, 1e-6))
        - _atan_pos(b2w / jnp.maximum(b2h, 1e-6))) ** 2
    alpha = v / jnp.maximum(1.0 - iou + v, 1e-6)
    return ciou - alpha * v


def _grid_body(x5, mt, mc, ms, loss_out, np_out):
    b = pl.program_id(0)
    a = pl.program_id(1)
    af = a.astype(jnp.float32)
    xl = x5[0, 0]
    yl = x5[0, 1]
    wl = x5[0, 2]
    hl = x5[0, 3]
    cl = x5[0, 4]
    px = jax.nn.sigmoid(xl)
    py = jax.nn.sigmoid(yl)
    conf = jax.nn.sigmoid(cl)
    aw = jnp.where(a == 0, _LVL[0, 0],
                   jnp.where(a == 1, _LVL[1, 0], _LVL[2, 0]))
    ah = jnp.where(a == 0, _LVL[0, 1],
                   jnp.where(a == 1, _LVL[1, 1], _LVL[2, 1]))
    col_i = _fiota((_H, _W), 1)
    row_i = _fiota((_H, _W), 0)
    pbx = px + col_i
    pby = py + row_i
    pbw = jnp.exp(wl) * aw
    pbh = jnp.exp(hl) * ah
    px1 = pbx - 0.5 * pbw
    px2 = pbx + 0.5 * pbw
    py1 = pby - 0.5 * pbh
    py2 = pby + 0.5 * pbh
    area_p = pbw * pbh

    m = mt[0]
    gx_r = m[0:1]
    gy_r = m[1:2]
    gw_r = m[2:3]
    gh_r = m[3:4]
    a_r = m[4:5]
    gi_r = m[5:6]
    gj_r = m[6:7]
    v_r = m[7:8]
    tw_r = m[9:10]
    th_r = m[10:11]
    sel_r = jnp.where(a_r == af, v_r, 0.0)
    mcol = mc[0]
    a_c = mcol[:, 4:5]
    gi_c = mcol[:, 5:6]
    gj_c = mcol[:, 6:7]
    v_c = mcol[:, 7:8]
    sel_c = jnp.where(a_c == af, v_c, 0.0)

    iota_hn = _fiota((_H, _NP), 0)
    amat = jnp.where(iota_hn == gj_r, sel_r, 0.0)
    iota_nw = _fiota((_NP, _W), 1)
    bt = jnp.where(iota_nw == gi_c, 1.0, 0.0)
    obj = jnp.minimum(
        jnp.dot(amat, bt, preferred_element_type=jnp.float32), 1.0)
    npos = jnp.sum(obj)

    ign = jnp.zeros((_H, _W), jnp.float32)
    for n in range(_N):
        gxn = ms[b, 0, n]
        gyn = ms[b, 1, n]
        gwn = ms[b, 2, n]
        ghn = ms[b, 3, n]
        bx1 = gxn - 0.5 * gwn
        bx2 = gxn + 0.5 * gwn
        by1 = gyn - 0.5 * ghn
        by2 = gyn + 0.5 * ghn
        iw = jnp.maximum(jnp.minimum(px2, bx2) - jnp.maximum(px1, bx1), 0.0)
        ih = jnp.maximum(jnp.minimum(py2, by2) - jnp.maximum(py1, by1), 0.0)
        inter3 = 3.0 * (iw * ih)
        ign = jnp.where(inter3 > area_p + gwn * ghn, 1.0, ign)
    no_obj = (1.0 - ign) * (1.0 - obj)
    loss_conf = jnp.sum(_bce_sum_terms(conf, obj) * (obj + no_obj))

    iota_wn = _fiota((_W, _NP), 0)
    bcol = jnp.where(iota_wn == gi_r, 1.0, 0.0)
    rowmask = jnp.where(iota_hn == gj_r, 1.0, 0.0)

    def gather(g):
        colsel = jnp.dot(g, bcol, preferred_element_type=jnp.float32)
        return jnp.sum(colsel * rowmask, axis=0, keepdims=True)

    gbx = gather(pbx)
    gby = gather(pby)
    gbw = gather(pbw)
    gbh = gather(pbh)

    same = jnp.where((gi_c == gi_r) & (gj_c == gj_r), sel_c * sel_r, 0.0)
    iota_m = jax.lax.broadcasted_iota(jnp.int32, (_NP, _NP), 0)
    iota_n = jax.lax.broadcasted_iota(jnp.int32, (_NP, _NP), 1)
    later = jnp.where(iota_m > iota_n, same, 0.0)
    overwritten = jnp.max(later, axis=0, keepdims=True)
    owner = sel_r * (1.0 - overwritten)

    ciou = _ciou(gbx, gby, gbw, gbh, gx_r, gy_r, gw_r, gh_r)
    scale = 2.0 - tw_r * th_r
    loss_loc = jnp.sum((1.0 - ciou) * scale * owner)

    val = (loss_conf + loss_loc) * (1.0 / 1024.0)
    loss_out[...] = jnp.full((1, 1, 8, 128), 0.0, jnp.float32) + val
    np_out[...] = jnp.full((1, 1, 8, 128), 0.0, jnp.float32) + npos * (1.0 / 1024.0)


def _cls_body(pf, xt_hbm, mt, mc, out, buf, sems):
    b = pl.program_id(0)

    def _issue(slot, bb):
        for n in range(_N):
            gj = pf[1, bb, n]
            gi = pf[2, bb, n]
            pltpu.make_async_copy(
                xt_hbm.at[gj, gi],
                buf.at[slot, n],
                sems.at[slot, n]).start()

    @pl.when(b == 0)
    def _():
        _issue(0, 0)

    @pl.when(b + 1 < _B)
    def _():
        _issue((b + 1) & 1, b + 1)

    m = mt[0]
    a_r = m[4:5]
    gi_r = m[5:6]
    gj_r = m[6:7]
    v_r = m[7:8]
    mcol = mc[0]
    a_c = mcol[:, 4:5]
    gi_c = mcol[:, 5:6]
    gj_c = mcol[:, 6:7]
    v_c = mcol[:, 7:8]

    same = jnp.where((a_c == a_r) & (gi_c == gi_r) & (gj_c == gj_r),
                     v_c * v_r, 0.0)
    iota_m = jax.lax.broadcasted_iota(jnp.int32, (_NP, _NP), 0)
    iota_n = jax.lax.broadcasted_iota(jnp.int32, (_NP, _NP), 1)
    later_t = jnp.where(iota_n > iota_m, same, 0.0)
    ow_c = jnp.max(later_t, axis=1, keepdims=True)
    owner_c = jnp.where(v_c > 0.0, 1.0 - ow_c, 0.0)

    same32 = same[0:32, 0:32]
    a32 = mcol[0:32, 4:5]
    cls32 = mcol[0:32, 8:9]
    ch32 = _CH * a32 + 5.0 + cls32
    iota_c = _fiota((32, 255), 1)
    oh = jnp.where(iota_c == ch32, 1.0, 0.0)
    cnt = jnp.dot(same32, oh, preferred_element_type=jnp.float32)
    multihot = jnp.minimum(cnt, 1.0)
    lo = _CH * a32 + 5.0
    chwin = jnp.where((iota_c >= lo) & (iota_c < lo + 80.0), 1.0, 0.0)

    iota_b = _fiota((_B, 1), 0)
    bmask = jnp.where(iota_b == b.astype(jnp.float32), 1.0, 0.0)
    slot = b & 1
    total = 0.0
    for n in range(_N):
        pltpu.make_async_copy(buf.at[slot, n], buf.at[slot, n],
                              sems.at[slot, n]).wait()
        blk = buf[slot, n]
        row = jnp.sum(blk * bmask, axis=0, keepdims=True)
        bce = _bce_sum_terms(jax.nn.sigmoid(row), multihot[n:n + 1])
        total = total + jnp.sum(bce * chwin[n:n + 1]) * owner_c[n, 0]
    out[...] = jnp.full((1, 8, 128), 0.0, jnp.float32) + total * (1.0 / 1024.0)


def _final_body(l1, n1, l2, loss_out, np_out):
    loss = jnp.sum(l1[...]) + jnp.sum(l2[...])
    npos = jnp.maximum(jnp.sum(n1[...]), 1.0)
    loss_out[...] = jnp.zeros((8, 128), jnp.float32) + loss
    np_out[...] = jnp.zeros((8, 128), jnp.float32) + npos


def kernel(input, targets):
    x = input
    t = targets.astype(jnp.float32)
    B, A, H, W, N = _B, _A, _H, _W, _N

    gx = t[..., 0] * W
    gy = t[..., 1] * H
    gw = t[..., 2] * W
    gh = t[..., 3] * H
    gi = jnp.floor(gx).astype(jnp.int32)
    gj = jnp.floor(gy).astype(jnp.int32)
    anw = jnp.asarray(_SC_ANCH[:, 0])
    anh = jnp.asarray(_SC_ANCH[:, 1])
    inter = jnp.minimum(gw[..., None], anw) * jnp.minimum(gh[..., None], anh)
    union = (gw * gh)[..., None] + anw * anh - inter
    best = jnp.argmax(inter / jnp.maximum(union, 1e-6), axis=-1)
    valid = (best >= _SUB) & (best < _SUB + A) & (gj < H) & (gi < W)
    a_idx = jnp.where(valid, best - _SUB, A).astype(jnp.int32)

    meta20 = jnp.stack([
        gx, gy, gw, gh,
        a_idx.astype(jnp.float32),
        gi.astype(jnp.float32), gj.astype(jnp.float32),
        valid.astype(jnp.float32),
        t[..., 4], t[..., 2], t[..., 3],
        (gj & 7).astype(jnp.float32),
    ], axis=1)
    meta_t = jnp.pad(meta20, ((0, 0), (0, 0), (0, _NP - N)))
    meta_c = jnp.transpose(meta_t, (0, 2, 1))

    a_safe = jnp.clip(a_idx, 0, A - 1)
    gj_safe = jnp.clip(gj, 0, H - 1)
    gi_safe = jnp.clip(gi, 0, W - 1)
    pf = jnp.stack([a_safe, gj_safe, gi_safe],
                   axis=0).astype(jnp.int32)
    xt = jnp.transpose(x, (2, 3, 0, 1))

    loss_p, np_p = pl.pallas_call(
        _grid_body,
        grid=(B, A),
        in_specs=[
            pl.BlockSpec((1, 5, H, W), lambda b, a: (b, 17 * a, 0, 0)),
            pl.BlockSpec((1, 12, _NP), lambda b, a: (b, 0, 0)),
            pl.BlockSpec((1, _NP, 12), lambda b, a: (b, 0, 0)),
            pl.BlockSpec(memory_space=pltpu.SMEM),
        ],
        out_specs=[
            pl.BlockSpec((1, 1, 8, 128), lambda b, a: (b, a, 0, 0)),
            pl.BlockSpec((1, 1, 8, 128), lambda b, a: (b, a, 0, 0)),
        ],
        out_shape=[
            jax.ShapeDtypeStruct((B, A, 8, 128), jnp.float32),
            jax.ShapeDtypeStruct((B, A, 8, 128), jnp.float32),
        ],
        compiler_params=pltpu.CompilerParams(
            dimension_semantics=("parallel", "arbitrary")),
        name="yolo_grid_pass",
        interpret=_INTERPRET,
    )(x, meta_t, meta_c, meta20)

    cls_p = pl.pallas_call(
        _cls_body,
        grid=(B,),
        in_specs=[
            pl.BlockSpec(memory_space=pltpu.SMEM),
            pl.BlockSpec(memory_space=pl.ANY),
            pl.BlockSpec((1, 12, _NP), lambda b: (b, 0, 0)),
            pl.BlockSpec((1, _NP, 12), lambda b: (b, 0, 0)),
        ],
        out_specs=pl.BlockSpec((1, 8, 128), lambda b: (b, 0, 0)),
        scratch_shapes=[
            pltpu.VMEM((2, _N, _B, 255), jnp.float32),
            pltpu.SemaphoreType.DMA((2, _N)),
        ],
        out_shape=jax.ShapeDtypeStruct((B, 8, 128), jnp.float32),
        compiler_params=pltpu.CompilerParams(
            dimension_semantics=("arbitrary",)),
        name="yolo_cls_pass",
        interpret=_INTERPRET,
    )(pf, xt, meta_t, meta_c)

    loss_o, np_o = pl.pallas_call(
        _final_body,
        out_shape=[
            jax.ShapeDtypeStruct((8, 128), jnp.float32),
            jax.ShapeDtypeStruct((8, 128), jnp.float32),
        ],
        name="yolo_finalize",
        interpret=_INTERPRET,
    )(loss_p, np_p, cls_p)

    return loss_o[0, 0], np_o[0, 0]
